# baseline (speedup 1.0000x reference)
import functools

import numpy as np
import jax
import jax.numpy as jnp
from jax import lax
from jax.experimental import pallas as pl
from jax.experimental.pallas import tpu as pltpu
from jax.experimental.pallas import tpu_sc as plsc

B, C, HW, P = 16, 3, 224, 16
GH = HW // P
K, D = 1024, 256
PD = C * P * P
N = B * GH * GH

TM = 392
NT = N // TM

NC, NS = 2, 16
NW = NC * NS

NCHUNK = PD // P
NROW = N * NCHUNK
RPW = NROW // NW
CH = 96
NDMA = RPW // CH


def _perm_tables():
    t = np.arange(N)
    b, gh, gw = t // (GH * GH), (t // GH) % GH, t % GH
    j = np.arange(NCHUNK)
    c, p1 = j // P, j % P
    perm = (((b[:, None] * C + c[None, :]) * HW + gh[:, None] * P + p1[None, :]) * GH
            + gw[:, None]).reshape(-1).astype(np.int32)
    r = np.arange(NROW)
    gw_r = r % GH
    q = r // GH
    h = q % HW
    bc = q // HW
    c_r, b_r = bc % C, bc // C
    gh_r, p1_r = h // P, h % P
    tok = (b_r * (GH * GH) + gh_r * GH + gw_r).astype(np.int32)
    offs = ((c_r * P + p1_r) * K).astype(np.int32)
    return perm, tok, offs


_PERM, _TOK, _OFFS = _perm_tables()


def _vq_body(z_ref, zsq_ref, cb_ref, wd_ref, bd_ref, idx_ref, tab_ref):
    @pl.when(pl.program_id(0) == 0)
    def _():
        tab_ref[...] = jnp.dot(cb_ref[...], wd_ref[...],
                               preferred_element_type=jnp.float32) + bd_ref[...]

    cbv = cb_ref[...]
    e_sq = jnp.sum(cbv * cbv, axis=1, keepdims=True)
    zc = lax.dot_general(z_ref[...], cbv, (((1,), (1,)), ((), ())),
                         precision=lax.Precision.DEFAULT,
                         preferred_element_type=jnp.float32)
    dist = (zsq_ref[...] + e_sq.T) - 2.0 * zc
    minv = jnp.min(dist, axis=1, keepdims=True)
    iota = lax.broadcasted_iota(jnp.int32, (TM, K), 1)
    idx_ref[0, 0, :] = jnp.min(jnp.where(dist == minv, iota, K), axis=1)


def _vq_and_table(z, z_sq, codebook, W_dec, b_dec):
    return pl.pallas_call(
        _vq_body,
        grid=(NT,),
        in_specs=[
            pl.BlockSpec((TM, D), lambda i: (i, 0)),
            pl.BlockSpec((TM, 1), lambda i: (i, 0)),
            pl.BlockSpec((K, D), lambda i: (0, 0)),
            pl.BlockSpec((D, PD), lambda i: (0, 0)),
            pl.BlockSpec((PD,), lambda i: (0,)),
        ],
        out_specs=[
            pl.BlockSpec((1, 1, TM), lambda i: (i, 0, 0)),
            pl.BlockSpec((K, PD), lambda i: (0, 0)),
        ],
        out_shape=[
            jax.ShapeDtypeStruct((NT, 1, TM), jnp.int32),
            jax.ShapeDtypeStruct((K, PD), jnp.float32),
        ],
    )(z, z_sq, codebook, W_dec, b_dec)


def _sc_gather(table2, src):
    mesh = plsc.VectorSubcoreMesh(core_axis_name="c", subcore_axis_name="s",
                                  num_cores=NC, num_subcores=NS)

    @functools.partial(
        pl.kernel,
        mesh=mesh,
        out_type=jax.ShapeDtypeStruct((NROW, P), jnp.float32),
        compiler_params=pltpu.CompilerParams(use_tc_tiling_on_sc=False),
        scratch_types=[
            pltpu.VMEM((RPW,), jnp.int32),
            pltpu.VMEM((RPW, P), jnp.float32),
            pltpu.SemaphoreType.DMA,
        ],
    )
    def gather(tab_hbm, src_hbm, out_hbm, idx_v, slab_v, sem):
        wid = lax.axis_index("s") * NC + lax.axis_index("c")
        base = wid * RPW
        pltpu.sync_copy(src_hbm.at[pl.ds(base, RPW)], idx_v)

        def body(j, carry):
            o = j * CH
            pltpu.async_copy(tab_hbm.at[idx_v.at[pl.ds(o, CH)]],
                             slab_v.at[pl.ds(o, CH)], sem)
            return carry

        lax.fori_loop(0, NDMA, body, 0)
        pltpu.make_async_copy(tab_hbm.at[pl.ds(0, RPW)], slab_v, sem).wait()
        pltpu.sync_copy(slab_v, out_hbm.at[pl.ds(base, RPW)])

    return gather(table2, src)


def kernel(x, codebook, W_enc, b_enc, W_dec, b_dec):
    x_rows = x.reshape(NROW, P)
    xp = _sc_gather(x_rows, _PERM).reshape(N, PD)
    z_e = xp @ W_enc + b_enc
    z_flat = z_e.reshape(N, D)
    z_sq = jnp.sum(z_flat ** 2, axis=1, keepdims=True)
    idx3, table = _vq_and_table(z_flat, z_sq, codebook, W_dec, b_dec)
    idx = idx3.reshape(N)
    table2 = table.reshape(K, NCHUNK, P).transpose(1, 0, 2).reshape(NCHUNK * K, P)
    src = _OFFS + idx[_TOK]
    x_recon = _sc_gather(table2, src).reshape(B, C, HW, HW)
    codes = idx.reshape(B, GH, GH)
    return x_recon, codes


# speedup vs baseline: 9.4619x; 9.4619x over previous
import functools

import numpy as np
import jax
import jax.numpy as jnp
from jax import lax
from jax.experimental import pallas as pl
from jax.experimental.pallas import tpu as pltpu
from jax.experimental.pallas import tpu_sc as plsc

B, C, HW, P = 16, 3, 224, 16
GH = HW // P
K, D = 1024, 256
PD = C * P * P
N = B * GH * GH

TM = 392
NT = N // TM

NC, NS = 2, 16
NW = NC * NS

NCHUNK = PD // P
NROW = N * NCHUNK
RPW = NROW // NW
CH = 96
NDMA = RPW // CH


def _perm_tables():
    t = np.arange(N)
    b, gh, gw = t // (GH * GH), (t // GH) % GH, t % GH
    j = np.arange(NCHUNK)
    c, p1 = j // P, j % P
    perm = (((b[:, None] * C + c[None, :]) * HW + gh[:, None] * P + p1[None, :]) * GH
            + gw[:, None]).reshape(-1).astype(np.int32)
    r = np.arange(NROW)
    gw_r = r % GH
    q = r // GH
    h = q % HW
    bc = q // HW
    c_r, b_r = bc % C, bc // C
    gh_r, p1_r = h // P, h % P
    tok = (b_r * (GH * GH) + gh_r * GH + gw_r).astype(np.int32)
    offs = ((c_r * P + p1_r) * K).astype(np.int32)
    return perm, tok, offs


_PERM, _TOK, _OFFS = _perm_tables()


def _vq_body(z_ref, zsq_ref, cb_ref, wd_ref, bd_ref, idx_ref, tab_ref):
    @pl.when(pl.program_id(0) == 0)
    def _():
        tab_ref[...] = jnp.dot(cb_ref[...], wd_ref[...],
                               preferred_element_type=jnp.float32) + bd_ref[...]

    cbv = cb_ref[...]
    e_sq = jnp.sum(cbv * cbv, axis=1, keepdims=True)
    zc = lax.dot_general(z_ref[...], cbv, (((1,), (1,)), ((), ())),
                         precision=lax.Precision.DEFAULT,
                         preferred_element_type=jnp.float32)
    dist = (zsq_ref[...] + e_sq.T) - 2.0 * zc
    minv = jnp.min(dist, axis=1, keepdims=True)
    iota = lax.broadcasted_iota(jnp.int32, (TM, K), 1)
    idx_ref[0, 0, :] = jnp.min(jnp.where(dist == minv, iota, K), axis=1)


def _vq_and_table(z, z_sq, codebook, W_dec, b_dec):
    return pl.pallas_call(
        _vq_body,
        grid=(NT,),
        in_specs=[
            pl.BlockSpec((TM, D), lambda i: (i, 0)),
            pl.BlockSpec((TM, 1), lambda i: (i, 0)),
            pl.BlockSpec((K, D), lambda i: (0, 0)),
            pl.BlockSpec((D, PD), lambda i: (0, 0)),
            pl.BlockSpec((PD,), lambda i: (0,)),
        ],
        out_specs=[
            pl.BlockSpec((1, 1, TM), lambda i: (i, 0, 0)),
            pl.BlockSpec((K, PD), lambda i: (0, 0)),
        ],
        out_shape=[
            jax.ShapeDtypeStruct((NT, 1, TM), jnp.int32),
            jax.ShapeDtypeStruct((K, PD), jnp.float32),
        ],
    )(z, z_sq, codebook, W_dec, b_dec)


def _sc_gather(table2, src):
    mesh = plsc.VectorSubcoreMesh(core_axis_name="c", subcore_axis_name="s",
                                  num_cores=NC, num_subcores=NS)

    @functools.partial(
        pl.kernel,
        mesh=mesh,
        out_type=jax.ShapeDtypeStruct((NROW, P), jnp.float32),
        compiler_params=pltpu.CompilerParams(use_tc_tiling_on_sc=False),
        scratch_types=[
            pltpu.VMEM((RPW,), jnp.int32),
            pltpu.VMEM((RPW, P), jnp.float32),
            pltpu.SemaphoreType.DMA,
        ],
    )
    def gather(tab_hbm, src_hbm, out_hbm, idx_v, slab_v, sem):
        wid = lax.axis_index("s") * NC + lax.axis_index("c")
        base = wid * RPW
        pltpu.sync_copy(src_hbm.at[pl.ds(base, RPW)], idx_v)

        def body(j, carry):
            o = j * CH
            pltpu.async_copy(tab_hbm.at[idx_v.at[pl.ds(o, CH)]],
                             slab_v.at[pl.ds(o, CH)], sem)
            return carry

        lax.fori_loop(0, NDMA, body, 0)
        pltpu.make_async_copy(tab_hbm.at[pl.ds(0, RPW)], slab_v, sem).wait()
        pltpu.sync_copy(slab_v, out_hbm.at[pl.ds(base, RPW)])

    return gather(table2, src)


def kernel(x, codebook, W_enc, b_enc, W_dec, b_dec):
    x_rows = x.reshape(NROW, P)
    xp = _sc_gather(x_rows, _PERM).reshape(N, PD)
    z_e = xp @ W_enc + b_enc
    z_flat = z_e.reshape(N, D)
    z_sq = jnp.sum(z_flat ** 2, axis=1, keepdims=True)
    idx3, table = _vq_and_table(z_flat, z_sq, codebook, W_dec, b_dec)
    idx = idx3.reshape(N)
    table2 = table.reshape(K, NCHUNK, P).transpose(1, 0, 2).reshape(NCHUNK * K, P)
    offs = (jnp.arange(NCHUNK, dtype=jnp.int32) * K)[:, None]
    src = (offs + idx[None, :]).reshape(C, P, B, GH, GH)
    src = src.transpose(2, 0, 3, 1, 4).reshape(NROW)
    x_recon = _sc_gather(table2, src).reshape(B, C, HW, HW)
    codes = idx.reshape(B, GH, GH)
    return x_recon, codes


# speedup vs baseline: 9.7854x; 1.0342x over previous
import functools

import numpy as np
import jax
import jax.numpy as jnp
from jax import lax
from jax.experimental import pallas as pl
from jax.experimental.pallas import tpu as pltpu
from jax.experimental.pallas import tpu_sc as plsc

B, C, HW, P = 16, 3, 224, 16
GH = HW // P
K, D = 1024, 256
PD = C * P * P
N = B * GH * GH

TM = 392
NT = N // TM

NC, NS = 2, 16
NW = NC * NS

NCHUNK = PD // P
NROW = N * NCHUNK
RPW = NROW // NW
CH = 96
NDMA = RPW // CH


def _perm_tables():
    t = np.arange(N)
    b, gh, gw = t // (GH * GH), (t // GH) % GH, t % GH
    j = np.arange(NCHUNK)
    c, p1 = j // P, j % P
    perm = (((b[:, None] * C + c[None, :]) * HW + gh[:, None] * P + p1[None, :]) * GH
            + gw[:, None]).reshape(-1).astype(np.int32)
    r = np.arange(NROW)
    gw_r = r % GH
    q = r // GH
    h = q % HW
    bc = q // HW
    c_r, b_r = bc % C, bc // C
    gh_r, p1_r = h // P, h % P
    tokloc = (gh_r * GH + gw_r).astype(np.int32)
    offs = ((c_r * P + p1_r) * K).astype(np.int32)
    return perm, tokloc, offs


_PERM, _TOKLOC, _OFFS = _perm_tables()


def _vq_body(z_ref, zsq_ref, cb_ref, wd_ref, bd_ref, idx_ref, tab_ref):
    @pl.when(pl.program_id(0) == 0)
    def _():
        tab_ref[...] = jnp.dot(cb_ref[...], wd_ref[...],
                               preferred_element_type=jnp.float32) + bd_ref[...]

    cbv = cb_ref[...]
    e_sq = jnp.sum(cbv * cbv, axis=1, keepdims=True)
    zc = lax.dot_general(z_ref[...], cbv, (((1,), (1,)), ((), ())),
                         precision=lax.Precision.DEFAULT,
                         preferred_element_type=jnp.float32)
    dist = (zsq_ref[...] + e_sq.T) - 2.0 * zc
    minv = jnp.min(dist, axis=1, keepdims=True)
    iota = lax.broadcasted_iota(jnp.int32, (TM, K), 1)
    idx_ref[0, 0, :] = jnp.min(jnp.where(dist == minv, iota, K), axis=1)


def _vq_and_table(z, z_sq, codebook, W_dec, b_dec):
    return pl.pallas_call(
        _vq_body,
        grid=(NT,),
        in_specs=[
            pl.BlockSpec((TM, D), lambda i: (i, 0)),
            pl.BlockSpec((TM, 1), lambda i: (i, 0)),
            pl.BlockSpec((K, D), lambda i: (0, 0)),
            pl.BlockSpec((D, PD), lambda i: (0, 0)),
            pl.BlockSpec((PD,), lambda i: (0,)),
        ],
        out_specs=[
            pl.BlockSpec((1, 1, TM), lambda i: (i, 0, 0)),
            pl.BlockSpec((K, PD), lambda i: (0, 0)),
        ],
        out_shape=[
            jax.ShapeDtypeStruct((NT, 1, TM), jnp.int32),
            jax.ShapeDtypeStruct((K, PD), jnp.float32),
        ],
    )(z, z_sq, codebook, W_dec, b_dec)


def _sc_decode_gather(table2, idx, tokloc, offs):
    mesh = plsc.VectorSubcoreMesh(core_axis_name="c", subcore_axis_name="s",
                                  num_cores=NC, num_subcores=NS)
    TPB = N // B
    IW = 200

    @functools.partial(
        pl.kernel,
        mesh=mesh,
        out_type=jax.ShapeDtypeStruct((NROW, P), jnp.float32),
        compiler_params=pltpu.CompilerParams(use_tc_tiling_on_sc=False,
                                             needs_layout_passes=False),
        scratch_types=[
            pltpu.VMEM((IW,), jnp.int32),
            pltpu.VMEM((RPW,), jnp.int32),
            pltpu.VMEM((RPW,), jnp.int32),
            pltpu.VMEM((RPW,), jnp.int32),
            pltpu.VMEM((RPW, P), jnp.float32),
            pltpu.SemaphoreType.DMA,
        ],
    )
    def gather(tab_hbm, idx_hbm, tok_hbm, off_hbm, out_hbm,
               idxw_v, tl_v, of_v, src_v, slab_v, sem):
        wid = lax.axis_index("s") * NC + lax.axis_index("c")
        base = wid * RPW
        bb = wid // 2
        al = (bb * TPB) % 8
        iw_start = pl.multiple_of(bb * TPB - al, 8)
        pltpu.sync_copy(idx_hbm.at[pl.ds(iw_start, IW)], idxw_v)
        pltpu.sync_copy(tok_hbm.at[pl.ds(base, RPW)], tl_v)
        pltpu.sync_copy(off_hbm.at[pl.ds(base, RPW)], of_v)

        def ibody(j, carry):
            o = j * 16
            t16 = tl_v[pl.ds(o, 16)] + al
            iv = plsc.load_gather(idxw_v, [t16])
            src_v[pl.ds(o, 16)] = of_v[pl.ds(o, 16)] + iv
            return carry

        lax.fori_loop(0, RPW // 16, ibody, 0)

        def body(j, carry):
            o = j * CH
            pltpu.async_copy(tab_hbm.at[src_v.at[pl.ds(o, CH)]],
                             slab_v.at[pl.ds(o, CH)], sem)
            return carry

        lax.fori_loop(0, NDMA, body, 0)
        pltpu.make_async_copy(tab_hbm.at[pl.ds(0, RPW)], slab_v, sem).wait()
        pltpu.sync_copy(slab_v, out_hbm.at[pl.ds(base, RPW)])

    return gather(table2, idx, tokloc, offs)


def _sc_gather(table2, src):
    mesh = plsc.VectorSubcoreMesh(core_axis_name="c", subcore_axis_name="s",
                                  num_cores=NC, num_subcores=NS)

    @functools.partial(
        pl.kernel,
        mesh=mesh,
        out_type=jax.ShapeDtypeStruct((NROW, P), jnp.float32),
        compiler_params=pltpu.CompilerParams(use_tc_tiling_on_sc=False),
        scratch_types=[
            pltpu.VMEM((RPW,), jnp.int32),
            pltpu.VMEM((RPW, P), jnp.float32),
            pltpu.SemaphoreType.DMA,
        ],
    )
    def gather(tab_hbm, src_hbm, out_hbm, idx_v, slab_v, sem):
        wid = lax.axis_index("s") * NC + lax.axis_index("c")
        base = wid * RPW
        pltpu.sync_copy(src_hbm.at[pl.ds(base, RPW)], idx_v)

        def body(j, carry):
            o = j * CH
            pltpu.async_copy(tab_hbm.at[idx_v.at[pl.ds(o, CH)]],
                             slab_v.at[pl.ds(o, CH)], sem)
            return carry

        lax.fori_loop(0, NDMA, body, 0)
        pltpu.make_async_copy(tab_hbm.at[pl.ds(0, RPW)], slab_v, sem).wait()
        pltpu.sync_copy(slab_v, out_hbm.at[pl.ds(base, RPW)])

    return gather(table2, src)


def kernel(x, codebook, W_enc, b_enc, W_dec, b_dec):
    x_rows = x.reshape(NROW, P)
    xp = _sc_gather(x_rows, _PERM).reshape(N, PD)
    z_e = xp @ W_enc + b_enc
    z_flat = z_e.reshape(N, D)
    z_sq = jnp.sum(z_flat ** 2, axis=1, keepdims=True)
    idx3, table = _vq_and_table(z_flat, z_sq, codebook, W_dec, b_dec)
    idx = idx3.reshape(N)
    table2 = table.reshape(K, NCHUNK, P).transpose(1, 0, 2).reshape(NCHUNK * K, P)
    x_recon = _sc_decode_gather(table2, idx, _TOKLOC, _OFFS).reshape(B, C, HW, HW)
    codes = idx.reshape(B, GH, GH)
    return x_recon, codes


# speedup vs baseline: 12.0916x; 1.2357x over previous
import functools

import numpy as np
import jax
import jax.numpy as jnp
from jax import lax
from jax.experimental import pallas as pl
from jax.experimental.pallas import tpu as pltpu
from jax.experimental.pallas import tpu_sc as plsc

B, C, HW, P = 16, 3, 224, 16
GH = HW // P
K, D = 1024, 256
PD = C * P * P
N = B * GH * GH

TM = 392
NT = N // TM

NC, NS = 2, 16
NW = NC * NS

NCHUNK = PD // P
NROW = N * NCHUNK
RPW = NROW // NW
CH = 96
NDMA = RPW // CH


def _perm_tables():
    t = np.arange(N)
    b, gh, gw = t // (GH * GH), (t // GH) % GH, t % GH
    j = np.arange(NCHUNK)
    c, p1 = j // P, j % P
    perm = (((b[:, None] * C + c[None, :]) * HW + gh[:, None] * P + p1[None, :]) * GH
            + gw[:, None]).reshape(-1).astype(np.int32)
    r = np.arange(NROW)
    gw_r = r % GH
    q = r // GH
    h = q % HW
    bc = q // HW
    c_r, b_r = bc % C, bc // C
    gh_r, p1_r = h // P, h % P
    tokloc = (gh_r * GH + gw_r).astype(np.int32)
    offs = (c_r * P + p1_r).astype(np.int32)
    return perm, tokloc, offs


_PERM, _TOKLOC, _OFFS = _perm_tables()


def _vq_body(z_ref, zsq_ref, cb_ref, wd_ref, bd_ref, idx_ref, tab_ref):
    @pl.when(pl.program_id(0) == 0)
    def _():
        tab_ref[...] = jnp.dot(cb_ref[...], wd_ref[...],
                               preferred_element_type=jnp.float32) + bd_ref[...]

    cbv = cb_ref[...]
    e_sq = jnp.sum(cbv * cbv, axis=1, keepdims=True)
    zc = lax.dot_general(z_ref[...], cbv, (((1,), (1,)), ((), ())),
                         precision=lax.Precision.DEFAULT,
                         preferred_element_type=jnp.float32)
    dist = (zsq_ref[...] + e_sq.T) - 2.0 * zc
    minv = jnp.min(dist, axis=1, keepdims=True)
    iota = lax.broadcasted_iota(jnp.int32, (TM, K), 1)
    idx_ref[0, 0, :] = jnp.min(jnp.where(dist == minv, iota, K), axis=1)


def _vq_and_table(z, z_sq, codebook, W_dec, b_dec):
    return pl.pallas_call(
        _vq_body,
        grid=(NT,),
        in_specs=[
            pl.BlockSpec((TM, D), lambda i: (i, 0)),
            pl.BlockSpec((TM, 1), lambda i: (i, 0)),
            pl.BlockSpec((K, D), lambda i: (0, 0)),
            pl.BlockSpec((D, PD), lambda i: (0, 0)),
            pl.BlockSpec((PD,), lambda i: (0,)),
        ],
        out_specs=[
            pl.BlockSpec((1, 1, TM), lambda i: (i, 0, 0)),
            pl.BlockSpec((K, PD), lambda i: (0, 0)),
        ],
        out_shape=[
            jax.ShapeDtypeStruct((NT, 1, TM), jnp.int32),
            jax.ShapeDtypeStruct((K, PD), jnp.float32),
        ],
    )(z, z_sq, codebook, W_dec, b_dec)


def _sc_decode_gather(table2, idx, tokloc, offs):
    mesh = plsc.VectorSubcoreMesh(core_axis_name="c", subcore_axis_name="s",
                                  num_cores=NC, num_subcores=NS)
    TPB = N // B
    IW = 200

    @functools.partial(
        pl.kernel,
        mesh=mesh,
        out_type=jax.ShapeDtypeStruct((NROW, P), jnp.float32),
        compiler_params=pltpu.CompilerParams(use_tc_tiling_on_sc=False,
                                             needs_layout_passes=False),
        scratch_types=[
            pltpu.VMEM((IW,), jnp.int32),
            pltpu.VMEM((RPW,), jnp.int32),
            pltpu.VMEM((RPW,), jnp.int32),
            pltpu.VMEM((RPW,), jnp.int32),
            pltpu.VMEM((RPW, P), jnp.float32),
            pltpu.SemaphoreType.DMA,
        ],
    )
    def gather(tab_hbm, idx_hbm, tok_hbm, off_hbm, out_hbm,
               idxw_v, tl_v, of_v, src_v, slab_v, sem):
        wid = lax.axis_index("s") * NC + lax.axis_index("c")
        base = wid * RPW
        bb = wid // 2
        al = (bb * TPB) % 8
        iw_start = pl.multiple_of(bb * TPB - al, 8)
        pltpu.sync_copy(idx_hbm.at[pl.ds(iw_start, IW)], idxw_v)
        pltpu.sync_copy(tok_hbm.at[pl.ds(base, RPW)], tl_v)
        pltpu.sync_copy(off_hbm.at[pl.ds(base, RPW)], of_v)

        def ibody(j, carry):
            o = j * 16
            t16 = tl_v[pl.ds(o, 16)] + al
            iv = plsc.load_gather(idxw_v, [t16])
            src_v[pl.ds(o, 16)] = iv * NCHUNK + of_v[pl.ds(o, 16)]
            return carry

        lax.fori_loop(0, RPW // 16, ibody, 0)

        def body(j, carry):
            o = j * CH
            pltpu.async_copy(tab_hbm.at[src_v.at[pl.ds(o, CH)]],
                             slab_v.at[pl.ds(o, CH)], sem)
            return carry

        lax.fori_loop(0, NDMA, body, 0)
        pltpu.make_async_copy(tab_hbm.at[pl.ds(0, RPW)], slab_v, sem).wait()
        pltpu.sync_copy(slab_v, out_hbm.at[pl.ds(base, RPW)])

    return gather(table2, idx, tokloc, offs)


def _sc_gather(table2, src):
    mesh = plsc.VectorSubcoreMesh(core_axis_name="c", subcore_axis_name="s",
                                  num_cores=NC, num_subcores=NS)

    @functools.partial(
        pl.kernel,
        mesh=mesh,
        out_type=jax.ShapeDtypeStruct((NROW, P), jnp.float32),
        compiler_params=pltpu.CompilerParams(use_tc_tiling_on_sc=False),
        scratch_types=[
            pltpu.VMEM((RPW,), jnp.int32),
            pltpu.VMEM((RPW, P), jnp.float32),
            pltpu.SemaphoreType.DMA,
        ],
    )
    def gather(tab_hbm, src_hbm, out_hbm, idx_v, slab_v, sem):
        wid = lax.axis_index("s") * NC + lax.axis_index("c")
        base = wid * RPW
        pltpu.sync_copy(src_hbm.at[pl.ds(base, RPW)], idx_v)

        def body(j, carry):
            o = j * CH
            pltpu.async_copy(tab_hbm.at[idx_v.at[pl.ds(o, CH)]],
                             slab_v.at[pl.ds(o, CH)], sem)
            return carry

        lax.fori_loop(0, NDMA, body, 0)
        pltpu.make_async_copy(tab_hbm.at[pl.ds(0, RPW)], slab_v, sem).wait()
        pltpu.sync_copy(slab_v, out_hbm.at[pl.ds(base, RPW)])

    return gather(table2, src)


def kernel(x, codebook, W_enc, b_enc, W_dec, b_dec):
    x_rows = x.reshape(NROW, P)
    xp = _sc_gather(x_rows, _PERM).reshape(N, PD)
    z_e = xp @ W_enc + b_enc
    z_flat = z_e.reshape(N, D)
    z_sq = jnp.sum(z_flat ** 2, axis=1, keepdims=True)
    idx3, table = _vq_and_table(z_flat, z_sq, codebook, W_dec, b_dec)
    idx = idx3.reshape(N)
    table2 = table.reshape(K * NCHUNK, P)
    x_recon = _sc_decode_gather(table2, idx, _TOKLOC, _OFFS).reshape(B, C, HW, HW)
    codes = idx.reshape(B, GH, GH)
    return x_recon, codes
